# Initial kernel scaffold; baseline (speedup 1.0000x reference)
import jax
import jax.numpy as jnp
from jax.experimental import pallas as pl


def kernel(x, edge_index, weight, batch, temporal_size, params):
    raise NotImplementedError("write your pallas kernel here")


# speedup vs baseline: 7.1260x; 7.1260x over previous
import jax
import jax.numpy as jnp
from jax.experimental import pallas as pl

_N = 50000
_E = 800000
_EMB = 64
_HID = 32
_T = 4
_B = 2

_BLK_N = 2000
_BLK_E = 8000
_BLK_R = 1600
_RPAD = 12800


def _embed_body(x_ref, w_ref, b_ref, o_ref):
    y = jnp.dot(x_ref[...], w_ref[...], preferred_element_type=jnp.float32)
    y = y + b_ref[...]
    o_ref[...] = jnp.where(y > 0, y, jnp.exp(y) - 1.0)


def _proj_body(h_ref, w_ref, o_ref):
    o_ref[...] = jnp.dot(h_ref[...], w_ref[...],
                         preferred_element_type=jnp.float32)


def _proj_elu_body(h_ref, w_ref, o_ref):
    h = h_ref[...]
    h = jnp.where(h > 0, h, jnp.exp(h) - 1.0)
    o_ref[...] = jnp.dot(h, w_ref[...], preferred_element_type=jnp.float32)


def _edge_e_body(a_ref, b_ref, o_ref):
    s = a_ref[...] + b_ref[...]
    o_ref[...] = jnp.where(s >= 0, s, 0.2 * s)


def _edge_exp_body(e_ref, m_ref, o_ref):
    o_ref[...] = jnp.exp(e_ref[...] - m_ref[...])


def _edge_msg_body(ee_ref, den_ref, w_ref, hp_ref, o_ref):
    coef = ee_ref[...] / (den_ref[...] + 1e-16) * w_ref[...]
    hp = hp_ref[...]
    m0 = hp[:, :_HID] * coef[:, 0:1]
    m1 = hp[:, _HID:] * coef[:, 1:2]
    o_ref[...] = jnp.concatenate([m0, m1], axis=1)


def _gru_body(x0_ref, x1_ref, x2_ref, x3_ref,
              wz_ref, uz_ref, bz_ref,
              wr_ref, ur_ref, br_ref,
              wn_ref, un_ref, bn_ref,
              wa_ref, ba_ref, va_ref,
              h_ref, s_ref):
    h = jnp.zeros((_BLK_R, _HID), jnp.float32)
    for x_ref in (x0_ref, x1_ref, x2_ref, x3_ref):
        xt = x_ref[...]
        tz = (jnp.dot(xt, wz_ref[...], preferred_element_type=jnp.float32)
              + jnp.dot(h, uz_ref[...], preferred_element_type=jnp.float32)
              + bz_ref[...])
        z = 1.0 / (1.0 + jnp.exp(-tz))
        tr = (jnp.dot(xt, wr_ref[...], preferred_element_type=jnp.float32)
              + jnp.dot(h, ur_ref[...], preferred_element_type=jnp.float32)
              + br_ref[...])
        r = 1.0 / (1.0 + jnp.exp(-tr))
        tn = (jnp.dot(xt, wn_ref[...], preferred_element_type=jnp.float32)
              + r * jnp.dot(h, un_ref[...], preferred_element_type=jnp.float32)
              + bn_ref[...])
        n = jnp.tanh(tn)
        h = (1.0 - z) * n + z * h
    h_ref[...] = h
    sa = jnp.tanh(jnp.dot(h, wa_ref[...], preferred_element_type=jnp.float32)
                  + ba_ref[...])
    s_ref[...] = jnp.dot(sa, va_ref[...], preferred_element_type=jnp.float32)


def _edge_call(body, out_width, *args):
    n_in = len(args)
    specs = []
    for a in args:
        w = a.shape[1]
        specs.append(pl.BlockSpec((_BLK_E, w), lambda i: (i, 0)))
    return pl.pallas_call(
        body,
        grid=(_E // _BLK_E,),
        in_specs=specs,
        out_specs=pl.BlockSpec((_BLK_E, out_width), lambda i: (i, 0)),
        out_shape=jax.ShapeDtypeStruct((_E, out_width), jnp.float32),
    )(*args)


def kernel(x, edge_index, weight, batch, temporal_size, params):
    p = params
    src = edge_index[0]
    dst = edge_index[1]
    wgt = weight.reshape(_E, 1)

    h = pl.pallas_call(
        _embed_body,
        grid=(_N // _BLK_N,),
        in_specs=[pl.BlockSpec((_BLK_N, 32), lambda i: (i, 0)),
                  pl.BlockSpec((32, _EMB), lambda i: (0, 0)),
                  pl.BlockSpec((1, _EMB), lambda i: (0, 0))],
        out_specs=pl.BlockSpec((_BLK_N, _EMB), lambda i: (i, 0)),
        out_shape=jax.ShapeDtypeStruct((_N, _EMB), jnp.float32),
    )(x, p['W_emb'], p['b_emb'].reshape(1, _EMB))

    layers = [(p['W1'], p['a1s'], p['a1d']),
              (p['W2'], p['a2s'], p['a2d']),
              (p['W3'], p['a3s'], p['a3d'])]

    for li, (W, a_s, a_d) in enumerate(layers):
        As = jnp.zeros((_EMB, 2), jnp.float32)
        As = As.at[0:_HID, 0].set(a_s[0]).at[_HID:, 1].set(a_s[1])
        Ad = jnp.zeros((_EMB, 2), jnp.float32)
        Ad = Ad.at[0:_HID, 0].set(a_d[0]).at[_HID:, 1].set(a_d[1])
        Wcat = jnp.concatenate([W, W @ As, W @ Ad], axis=1)

        body = _proj_body if li == 0 else _proj_elu_body
        out = pl.pallas_call(
            body,
            grid=(_N // _BLK_N,),
            in_specs=[pl.BlockSpec((_BLK_N, _EMB), lambda i: (i, 0)),
                      pl.BlockSpec((_EMB, 68), lambda i: (0, 0))],
            out_specs=pl.BlockSpec((_BLK_N, 68), lambda i: (i, 0)),
            out_shape=jax.ShapeDtypeStruct((_N, 68), jnp.float32),
        )(h, Wcat)
        hp = out[:, :_EMB]
        es = out[:, _EMB:_EMB + 2]
        ed = out[:, _EMB + 2:_EMB + 4]

        e = _edge_call(_edge_e_body, 2,
                       jnp.take(es, src, axis=0), jnp.take(ed, dst, axis=0))
        emax = jax.ops.segment_max(e, dst, num_segments=_N)
        emax = jnp.where(jnp.isfinite(emax), emax, 0.0)
        ee = _edge_call(_edge_exp_body, 2, e, jnp.take(emax, dst, axis=0))
        den = jax.ops.segment_sum(ee, dst, num_segments=_N)
        msg = _edge_call(_edge_msg_body, _EMB,
                         ee, jnp.take(den, dst, axis=0), wgt,
                         jnp.take(hp, src, axis=0))
        h = jax.ops.segment_sum(msg, dst, num_segments=_N)

    L = _N // (_B * _T)
    node_emb = h.reshape(_B, _T, L, _EMB)
    seq = jnp.transpose(node_emb, (0, 2, 1, 3)).reshape(_B * L, _T, _EMB)
    pad = _RPAD - _B * L
    xts = [jnp.pad(seq[:, t], ((0, pad), (0, 0))) for t in range(_T)]

    full = lambda shape: pl.BlockSpec(shape, lambda i: (0, 0))
    hstate, s = pl.pallas_call(
        _gru_body,
        grid=(_RPAD // _BLK_R,),
        in_specs=[pl.BlockSpec((_BLK_R, _EMB), lambda i: (i, 0))] * 4 + [
            full((_EMB, _HID)), full((_HID, _HID)), full((1, _HID)),
            full((_EMB, _HID)), full((_HID, _HID)), full((1, _HID)),
            full((_EMB, _HID)), full((_HID, _HID)), full((1, _HID)),
            full((_HID, _HID)), full((1, _HID)), full((_HID, 1)),
        ],
        out_specs=[pl.BlockSpec((_BLK_R, _HID), lambda i: (i, 0)),
                   pl.BlockSpec((_BLK_R, 1), lambda i: (i, 0))],
        out_shape=[jax.ShapeDtypeStruct((_RPAD, _HID), jnp.float32),
                   jax.ShapeDtypeStruct((_RPAD, 1), jnp.float32)],
    )(*xts,
      p['Wz'], p['Uz'], p['bz'].reshape(1, _HID),
      p['Wr'], p['Ur'], p['br'].reshape(1, _HID),
      p['Wn'], p['Un'], p['bn'].reshape(1, _HID),
      p['Wa'], p['ba'].reshape(1, _HID), p['va'].reshape(_HID, 1))

    temporal = hstate[:_B * L].reshape(_B, L, _HID)
    svec = s[:_B * L, 0].reshape(_B, L)
    w = jax.nn.softmax(svec, axis=-1)
    agg = jnp.sum(w[..., None] * temporal, axis=1)
    hm = jax.nn.gelu(agg @ p['Wh1'] + p['bh1'])
    val = hm @ p['Wh2'] + p['bh2']
    return val.squeeze(-1)
